# Initial kernel scaffold; baseline (speedup 1.0000x reference)
import jax
import jax.numpy as jnp
from jax.experimental import pallas as pl


def kernel(boxes, scores):
    raise NotImplementedError("write your pallas kernel here")


# speedup vs baseline: 29.5417x; 29.5417x over previous
import jax
import jax.numpy as jnp
from jax.experimental import pallas as pl
from jax.experimental.pallas import tpu as pltpu

PRE_K = 2000
KP = 2048
POST_K = 1000
THRESH = 0.7
B = 128
NB = KP // B


def _nms_keep_kernel(x1c_ref, y1c_ref, x2c_ref, y2c_ref,
                     x1r_ref, y1r_ref, x2r_ref, y2r_ref,
                     keep_ref, s_ref):
    x1c = x1c_ref[:, :]
    y1c = y1c_ref[:, :]
    x2c = x2c_ref[:, :]
    y2c = y2c_ref[:, :]
    area_c = (x2c - x1c) * (y2c - y1c)
    col_ids = jax.lax.broadcasted_iota(jnp.int32, (1, KP), 1)

    def outer(bi, keep):
        base = bi * B
        x1r = x1r_ref[pl.ds(base, B), :]
        y1r = y1r_ref[pl.ds(base, B), :]
        x2r = x2r_ref[pl.ds(base, B), :]
        y2r = y2r_ref[pl.ds(base, B), :]
        area_r = (x2r - x1r) * (y2r - y1r)
        xx1 = jnp.maximum(x1r, x1c)
        yy1 = jnp.maximum(y1r, y1c)
        xx2 = jnp.minimum(x2r, x2c)
        yy2 = jnp.minimum(y2r, y2c)
        iw = jnp.clip(xx2 - xx1, 0.0)
        ih = jnp.clip(yy2 - yy1, 0.0)
        inter = iw * ih
        union = area_r + area_c - inter
        iou = inter / (union + 1e-9)
        s_ref[:, :] = jnp.where(iou > THRESH, 1.0, 0.0)

        def inner(i, keep_in):
            gi = base + i
            ki = jnp.max(jnp.where(col_ids == gi, keep_in, 0.0))
            row = s_ref[pl.ds(i, 1), :]
            future = (col_ids > gi).astype(jnp.float32)
            return keep_in * (1.0 - ki * row * future)

        return jax.lax.fori_loop(0, B, inner, keep)

    keep = jnp.ones((1, KP), jnp.float32)
    keep = jax.lax.fori_loop(0, NB, outer, keep)
    keep_ref[:, :] = keep


def _nms_keep(x1, y1, x2, y2):
    c = lambda v: v.reshape(1, KP)
    r = lambda v: v.reshape(KP, 1)
    keep_f = pl.pallas_call(
        _nms_keep_kernel,
        out_shape=jax.ShapeDtypeStruct((1, KP), jnp.float32),
        scratch_shapes=[pltpu.VMEM((B, KP), jnp.float32)],
    )(c(x1), c(y1), c(x2), c(y2), r(x1), r(y1), r(x2), r(y2))
    return keep_f[0, :PRE_K] > 0.5


def kernel(boxes, scores):
    x1 = jnp.minimum(boxes[:, 0], boxes[:, 2])
    y1 = jnp.minimum(boxes[:, 1], boxes[:, 3])
    x2 = jnp.maximum(boxes[:, 0], boxes[:, 2])
    y2 = jnp.maximum(boxes[:, 1], boxes[:, 3])
    valid = ((x2 - x1) > 0.0) & ((y2 - y1) > 0.0)
    scores_m = jnp.where(valid, scores, -1e30)
    top_scores, idx = jax.lax.top_k(scores_m, PRE_K)

    bx1 = jnp.take(x1, idx)
    by1 = jnp.take(y1, idx)
    bx2 = jnp.take(x2, idx)
    by2 = jnp.take(y2, idx)

    pad = KP - PRE_K
    keep = _nms_keep(jnp.pad(bx1, (0, pad)), jnp.pad(by1, (0, pad)),
                     jnp.pad(bx2, (0, pad)), jnp.pad(by2, (0, pad)))

    masked = jnp.where(keep, top_scores, -1e30)
    _, sel = jax.lax.top_k(masked, POST_K)
    b = jnp.stack([bx1, by1, bx2, by2], axis=1)
    out_boxes = jnp.take(b, sel, axis=0)
    out_scores = jnp.where(jnp.take(keep, sel), jnp.take(top_scores, sel), 0.0)
    return jnp.concatenate([out_boxes, out_scores[:, None]], axis=1)


# speedup vs baseline: 31.5101x; 1.0666x over previous
import jax
import jax.numpy as jnp
from jax.experimental import pallas as pl
from jax.experimental.pallas import tpu as pltpu

PRE_K = 2000
KP = 2048
POST_K = 1000
THRESH = 0.7
B = 128
NB = KP // B


def _nms_keep_kernel(x1r_ref, y1r_ref, x2r_ref, y2r_ref,
                     x1q_ref, y1q_ref, x2q_ref, y2q_ref,
                     keep_ref, s_ref, sbb_ref):
    x1q = x1q_ref[:, :, :]
    y1q = y1q_ref[:, :, :]
    x2q = x2q_ref[:, :, :]
    y2q = y2q_ref[:, :, :]
    area_q = (x2q - x1q) * (y2q - y1q)

    blk_ids = jax.lax.broadcasted_iota(jnp.int32, (NB, B), 0)
    lane_ids = jax.lax.broadcasted_iota(jnp.int32, (1, B), 1)
    eye = (jax.lax.broadcasted_iota(jnp.int32, (B, B), 0) ==
           jax.lax.broadcasted_iota(jnp.int32, (B, B), 1))

    def outer(bi, keep):
        base = bi * B
        x1r = x1r_ref[pl.ds(base, B)]
        y1r = y1r_ref[pl.ds(base, B)]
        x2r = x2r_ref[pl.ds(base, B)]
        y2r = y2r_ref[pl.ds(base, B)]
        area_r = (x2r - x1r) * (y2r - y1r)
        xx1 = jnp.maximum(x1r, x1q)
        yy1 = jnp.maximum(y1r, y1q)
        xx2 = jnp.minimum(x2r, x2q)
        yy2 = jnp.minimum(y2r, y2q)
        iw = jnp.clip(xx2 - xx1, 0.0)
        ih = jnp.clip(yy2 - yy1, 0.0)
        inter = iw * ih
        union = area_r + area_q - inter
        iou = inter / (union + 1e-9)
        s_ref[:, :, :] = jnp.where(iou > THRESH, 1.0, 0.0)
        sbb_ref[:, :] = s_ref[:, pl.ds(bi, 1), :].reshape(B, B)

        kb0 = jnp.max(jnp.where(blk_ids == bi, keep, 0.0), axis=0,
                      keepdims=True)

        def inner(i, kb):
            ki = jnp.max(jnp.where(lane_ids == i, kb, 0.0))
            row = sbb_ref[pl.ds(i, 1), :]
            fut = (lane_ids > i).astype(jnp.float32)
            return kb * (1.0 - ki * row * fut)

        kb = jax.lax.fori_loop(0, B, inner, kb0)

        kbr = jnp.max(jnp.where(eye, kb, 0.0), axis=1, keepdims=True)
        sup = jnp.max(s_ref[:, :, :] * kbr.reshape(B, 1, 1), axis=0)
        future = (blk_ids > bi).astype(jnp.float32)
        keep = keep * (1.0 - sup * future)
        keep = jnp.where(blk_ids == bi, kb, keep)
        return keep

    keep = jnp.ones((NB, B), jnp.float32)
    keep = jax.lax.fori_loop(0, NB, outer, keep)
    keep_ref[:, :] = keep


def _nms_keep(x1, y1, x2, y2):
    r = lambda v: v.reshape(KP, 1, 1)
    q = lambda v: v.reshape(1, NB, B)
    keep_f = pl.pallas_call(
        _nms_keep_kernel,
        out_shape=jax.ShapeDtypeStruct((NB, B), jnp.float32),
        scratch_shapes=[pltpu.VMEM((B, NB, B), jnp.float32),
                        pltpu.VMEM((B, B), jnp.float32)],
    )(r(x1), r(y1), r(x2), r(y2), q(x1), q(y1), q(x2), q(y2))
    return keep_f.reshape(KP)[:PRE_K] > 0.5


def kernel(boxes, scores):
    x1 = jnp.minimum(boxes[:, 0], boxes[:, 2])
    y1 = jnp.minimum(boxes[:, 1], boxes[:, 3])
    x2 = jnp.maximum(boxes[:, 0], boxes[:, 2])
    y2 = jnp.maximum(boxes[:, 1], boxes[:, 3])
    valid = ((x2 - x1) > 0.0) & ((y2 - y1) > 0.0)
    scores_m = jnp.where(valid, scores, -1e30)
    top_scores, idx = jax.lax.top_k(scores_m, PRE_K)

    bx1 = jnp.take(x1, idx)
    by1 = jnp.take(y1, idx)
    bx2 = jnp.take(x2, idx)
    by2 = jnp.take(y2, idx)

    pad = KP - PRE_K
    keep = _nms_keep(jnp.pad(bx1, (0, pad)), jnp.pad(by1, (0, pad)),
                     jnp.pad(bx2, (0, pad)), jnp.pad(by2, (0, pad)))

    masked = jnp.where(keep, top_scores, -1e30)
    _, sel = jax.lax.top_k(masked, POST_K)
    b = jnp.stack([bx1, by1, bx2, by2], axis=1)
    out_boxes = jnp.take(b, sel, axis=0)
    out_scores = jnp.where(jnp.take(keep, sel), jnp.take(top_scores, sel), 0.0)
    return jnp.concatenate([out_boxes, out_scores[:, None]], axis=1)


# speedup vs baseline: 40.9788x; 1.3005x over previous
import jax
import jax.numpy as jnp
from jax.experimental import pallas as pl
from jax.experimental.pallas import tpu as pltpu

PRE_K = 2000
KP = 2048
POST_K = 1000
THRESH = 0.7
B = 128
NB = KP // B


def _nms_keep_kernel(x1r_ref, y1r_ref, x2r_ref, y2r_ref,
                     x1q_ref, y1q_ref, x2q_ref, y2q_ref,
                     keep_ref, s_ref, sbb_ref):
    x1q = x1q_ref[:, :, :]
    y1q = y1q_ref[:, :, :]
    x2q = x2q_ref[:, :, :]
    y2q = y2q_ref[:, :, :]
    area_q = (x2q - x1q) * (y2q - y1q)

    blk_ids = jax.lax.broadcasted_iota(jnp.int32, (NB, B), 0)
    lane_ids = jax.lax.broadcasted_iota(jnp.int32, (1, B), 1)
    row_bb = jax.lax.broadcasted_iota(jnp.int32, (B, B), 0)
    col_bb = jax.lax.broadcasted_iota(jnp.int32, (B, B), 1)
    eye = row_bb == col_bb
    upper = (col_bb > row_bb).astype(jnp.float32)

    def outer(bi, keep):
        base = bi * B
        x1r = x1r_ref[pl.ds(base, B)]
        y1r = y1r_ref[pl.ds(base, B)]
        x2r = x2r_ref[pl.ds(base, B)]
        y2r = y2r_ref[pl.ds(base, B)]
        area_r = (x2r - x1r) * (y2r - y1r)
        xx1 = jnp.maximum(x1r, x1q)
        yy1 = jnp.maximum(y1r, y1q)
        xx2 = jnp.minimum(x2r, x2q)
        yy2 = jnp.minimum(y2r, y2q)
        iw = jnp.clip(xx2 - xx1, 0.0)
        ih = jnp.clip(yy2 - yy1, 0.0)
        inter = iw * ih
        union = area_r + area_q - inter
        iou = inter / (union + 1e-9)
        s_ref[:, :, :] = jnp.where(iou > THRESH, 1.0, 0.0)
        sbb_ref[:, :] = s_ref[:, pl.ds(bi, 1), :].reshape(B, B) * upper

        kb0 = jnp.max(jnp.where(blk_ids == bi, keep, 0.0), axis=0,
                      keepdims=True)

        def inner(i, kb):
            ki = jnp.max(jnp.where(lane_ids == i, kb, 0.0), axis=1,
                         keepdims=True)
            return kb * (1.0 - ki * sbb_ref[pl.ds(i, 1), :])

        kb = jax.lax.fori_loop(0, B, inner, kb0, unroll=4)

        kbr = jnp.max(jnp.where(eye, kb, 0.0), axis=1, keepdims=True)
        sup = jnp.max(s_ref[:, :, :] * kbr.reshape(B, 1, 1), axis=0)
        future = (blk_ids > bi).astype(jnp.float32)
        keep = keep * (1.0 - sup * future)
        keep = jnp.where(blk_ids == bi, kb, keep)
        return keep

    keep = jnp.ones((NB, B), jnp.float32)
    keep = jax.lax.fori_loop(0, NB, outer, keep)
    keep_ref[:, :] = keep


def _nms_keep(x1, y1, x2, y2):
    r = lambda v: v.reshape(KP, 1, 1)
    q = lambda v: v.reshape(1, NB, B)
    keep_f = pl.pallas_call(
        _nms_keep_kernel,
        out_shape=jax.ShapeDtypeStruct((NB, B), jnp.float32),
        scratch_shapes=[pltpu.VMEM((B, NB, B), jnp.float32),
                        pltpu.VMEM((B, B), jnp.float32)],
    )(r(x1), r(y1), r(x2), r(y2), q(x1), q(y1), q(x2), q(y2))
    return keep_f.reshape(KP)[:PRE_K] > 0.5


def kernel(boxes, scores):
    x1 = jnp.minimum(boxes[:, 0], boxes[:, 2])
    y1 = jnp.minimum(boxes[:, 1], boxes[:, 3])
    x2 = jnp.maximum(boxes[:, 0], boxes[:, 2])
    y2 = jnp.maximum(boxes[:, 1], boxes[:, 3])
    valid = ((x2 - x1) > 0.0) & ((y2 - y1) > 0.0)
    scores_m = jnp.where(valid, scores, -1e30)
    top_scores, idx = jax.lax.top_k(scores_m, PRE_K)

    bx1 = jnp.take(x1, idx)
    by1 = jnp.take(y1, idx)
    bx2 = jnp.take(x2, idx)
    by2 = jnp.take(y2, idx)

    pad = KP - PRE_K
    keep = _nms_keep(jnp.pad(bx1, (0, pad)), jnp.pad(by1, (0, pad)),
                     jnp.pad(bx2, (0, pad)), jnp.pad(by2, (0, pad)))

    masked = jnp.where(keep, top_scores, -1e30)
    _, sel = jax.lax.top_k(masked, POST_K)
    b = jnp.stack([bx1, by1, bx2, by2], axis=1)
    out_boxes = jnp.take(b, sel, axis=0)
    out_scores = jnp.where(jnp.take(keep, sel), jnp.take(top_scores, sel), 0.0)
    return jnp.concatenate([out_boxes, out_scores[:, None]], axis=1)
